# baseline (speedup 1.0000x reference)
import functools

import jax
import jax.numpy as jnp
from jax.experimental import pallas as pl
from jax.experimental.pallas import tpu as pltpu

HIGHEST = jax.lax.Precision.HIGHEST


def _router_kernel(x_ref, w_ref, b_ref, g_ref, counts_ref, psum_ref, nd_ref,
                   *, n_experts, capacity):
    xv = x_ref[...]
    T = xv.shape[0]
    logits = jnp.dot(xv.astype(jnp.bfloat16), w_ref[...].astype(jnp.bfloat16),
                     preferred_element_type=jnp.float32) + b_ref[...]
    mx = jnp.max(logits, axis=1, keepdims=True)
    el = jnp.exp(logits - mx)
    probs = el / jnp.sum(el, axis=1, keepdims=True)
    psum_ref[...] = jnp.sum(probs, axis=0, keepdims=True)

    eidx = jax.lax.broadcasted_iota(jnp.int32, (T, n_experts), 1)
    is_max = logits == mx
    route = jnp.min(jnp.where(is_max, eidx, n_experts), axis=1, keepdims=True)

    onehot = (eidx == route).astype(jnp.float32)
    counts_ref[...] = jnp.sum(onehot, axis=0, keepdims=True)

    r_io = jax.lax.broadcasted_iota(jnp.int32, (T, T), 0)
    c_io = jax.lax.broadcasted_iota(jnp.int32, (T, T), 1)
    tril = (r_io >= c_io).astype(jnp.bfloat16)
    csum = jnp.dot(tril, onehot.astype(jnp.bfloat16),
                   preferred_element_type=jnp.float32)
    pos = jnp.sum((csum - 1.0) * onehot, axis=1, keepdims=True)
    pos_i = jnp.round(pos).astype(jnp.int32)
    keep = pos_i < capacity
    g_ref[...] = jnp.where(keep, route * capacity + pos_i,
                           n_experts * capacity)
    nd_ref[...] = T - jnp.sum(keep.astype(jnp.int32), keepdims=True)


def _expert_kernel(g_ref, x_ref, w1_ref, b1_ref, w2_ref, b2_ref, y_ref,
                   buf_ref, *, capacity, n_ff_chunks):
    e = pl.program_id(0)
    f = pl.program_id(1)

    @pl.when(f == 0)
    def _():
        gv = g_ref[...]
        cidx = jax.lax.broadcasted_iota(jnp.int32, (capacity, gv.shape[1]), 0)
        m = (gv == e * capacity + cidx).astype(jnp.bfloat16)
        buf_ref[...] = jnp.dot(m, x_ref[...],
                               preferred_element_type=jnp.float32)

    h = jnp.dot(buf_ref[...], w1_ref[...],
                preferred_element_type=jnp.float32) + b1_ref[...]
    h = 0.5 * h * (1.0 + jax.lax.erf(h * 0.7071067811865476))
    part = jnp.dot(h, w2_ref[...], preferred_element_type=jnp.float32)

    @pl.when(f == 0)
    def _():
        y_ref[...] = part + b2_ref[...]

    @pl.when(f != 0)
    def _():
        y_ref[...] += part


def _combine_kernel(g_ref, x_ref, y_ref, out_ref, *, n_slots):
    gcol = g_ref[...]
    sidx = jax.lax.broadcasted_iota(jnp.int32, (gcol.shape[0], n_slots), 1)
    onehot = (gcol == sidx).astype(jnp.bfloat16)
    gathered = jnp.dot(onehot, y_ref[...].astype(jnp.bfloat16),
                       preferred_element_type=jnp.float32)
    keepf = (gcol < n_slots).astype(jnp.float32)
    out_ref[...] = gathered + (1.0 - keepf) * x_ref[...]


def kernel(x, Wsw, bsw, W1, b1, W2, b2):
    b, s, d = x.shape
    T = b * s
    E = Wsw.shape[1]
    FF = W1.shape[2]
    C = int(T * 1.25 / E)
    n_slots = E * C

    xt = x.reshape(T, d)

    g_col, counts, psum, nd = pl.pallas_call(
        functools.partial(_router_kernel, n_experts=E, capacity=C),
        out_shape=(
            jax.ShapeDtypeStruct((T, 1), jnp.int32),
            jax.ShapeDtypeStruct((1, E), jnp.float32),
            jax.ShapeDtypeStruct((1, E), jnp.float32),
            jax.ShapeDtypeStruct((1, 1), jnp.int32),
        ),
    )(xt, Wsw, bsw.reshape(1, E))

    g_row = g_col.reshape(1, T)
    xtb = xt.astype(jnp.bfloat16)

    n_ff_chunks = 4
    ffc = FF // n_ff_chunks
    y_flat = pl.pallas_call(
        functools.partial(_expert_kernel, capacity=C, n_ff_chunks=n_ff_chunks),
        grid=(E, n_ff_chunks),
        in_specs=[
            pl.BlockSpec((1, T), lambda e, f: (0, 0)),
            pl.BlockSpec((T, d), lambda e, f: (0, 0)),
            pl.BlockSpec((None, d, ffc), lambda e, f: (e, 0, f)),
            pl.BlockSpec((None, 1, ffc), lambda e, f: (e, 0, f)),
            pl.BlockSpec((None, ffc, d), lambda e, f: (e, f, 0)),
            pl.BlockSpec((None, 1, d), lambda e, f: (e, 0, 0)),
        ],
        out_specs=pl.BlockSpec((None, C, d), lambda e, f: (e, 0, 0)),
        out_shape=jax.ShapeDtypeStruct((E, C, d), jnp.float32),
        scratch_shapes=[pltpu.VMEM((C, d), jnp.float32)],
    )(g_row, xtb, W1, b1.reshape(E, 1, FF), W2, b2.reshape(E, 1, d))

    y2 = y_flat.reshape(n_slots, d)

    tb = 256
    out_t = pl.pallas_call(
        functools.partial(_combine_kernel, n_slots=n_slots),
        grid=(T // tb,),
        in_specs=[
            pl.BlockSpec((tb, 1), lambda i: (i, 0)),
            pl.BlockSpec((tb, d), lambda i: (i, 0)),
            pl.BlockSpec((n_slots, d), lambda i: (0, 0)),
        ],
        out_specs=pl.BlockSpec((tb, d), lambda i: (i, 0)),
        out_shape=jax.ShapeDtypeStruct((T, d), jnp.float32),
    )(g_col, xt, y2)

    out = out_t.reshape(b, s, d)
    return (out, counts.reshape(E), psum.reshape(E), nd.reshape(()))


# speedup vs baseline: 1.1927x; 1.1927x over previous
import functools

import jax
import jax.numpy as jnp
from jax import lax
from jax.experimental import pallas as pl
from jax.experimental.pallas import tpu as pltpu
from jax.experimental.pallas import tpu_sc as plsc


def _router_kernel(x_ref, w_ref, b_ref, g_ref, counts_ref, psum_ref, nd_ref,
                   *, n_experts, capacity, row_pitch, x_chunk):
    xv = x_ref[...]
    T = xv.shape[0]
    logits = jnp.dot(xv, w_ref[...].astype(jnp.bfloat16),
                     preferred_element_type=jnp.float32) + b_ref[...]
    mx = jnp.max(logits, axis=1, keepdims=True)
    el = jnp.exp(logits - mx)
    probs = el / jnp.sum(el, axis=1, keepdims=True)
    psum_ref[...] = jnp.sum(probs, axis=0, keepdims=True)

    eidx = lax.broadcasted_iota(jnp.int32, (T, n_experts), 1)
    is_max = logits == mx
    route = jnp.min(jnp.where(is_max, eidx, n_experts), axis=1, keepdims=True)

    onehot = (eidx == route).astype(jnp.float32)
    counts_ref[...] = jnp.sum(onehot, axis=0, keepdims=True)

    r_io = lax.broadcasted_iota(jnp.int32, (T, T), 0)
    c_io = lax.broadcasted_iota(jnp.int32, (T, T), 1)
    tril = (r_io >= c_io).astype(jnp.bfloat16)
    csum = jnp.dot(tril, onehot.astype(jnp.bfloat16),
                   preferred_element_type=jnp.float32)
    pos = jnp.sum((csum - 1.0) * onehot, axis=1, keepdims=True)
    pos_i = jnp.round(pos).astype(jnp.int32)
    keep = pos_i < capacity
    tk = lax.broadcasted_iota(jnp.int32, (T, 1), 0)
    g_ref[...] = jnp.where(
        keep,
        route * row_pitch + pos_i,
        (tk // x_chunk) * row_pitch + capacity + tk % x_chunk)
    nd_ref[...] = T - jnp.sum(keep.astype(jnp.int32), keepdims=True)


def _expert_kernel(g_ref, x_ref, xf_ref, w1_ref, b1_ref, w2_ref, b2_ref,
                   y_ref, *, capacity, row_pitch):
    e = pl.program_id(0)
    gv = g_ref[...]
    cidx = lax.broadcasted_iota(jnp.int32, (capacity, gv.shape[1]), 0)
    m = (gv == e * row_pitch + cidx).astype(jnp.bfloat16)
    buf = jnp.dot(m, x_ref[...], preferred_element_type=jnp.float32)

    h = jnp.dot(buf, w1_ref[...],
                preferred_element_type=jnp.float32) + b1_ref[...]
    h = 0.5 * h * (1.0 + lax.erf(h * 0.7071067811865476))
    y_ref[0:capacity, :] = (jnp.dot(h, w2_ref[...],
                                    preferred_element_type=jnp.float32)
                            + b2_ref[...])
    y_ref[capacity:row_pitch, :] = xf_ref[...]


def _make_sc_combine(T, d, n_workers):
    rows_per_w = T // n_workers
    mesh = plsc.VectorSubcoreMesh(core_axis_name="c", subcore_axis_name="s")

    @functools.partial(
        pl.kernel, mesh=mesh,
        out_type=jax.ShapeDtypeStruct((T, d), jnp.float32),
        scratch_types=[
            pltpu.VMEM((rows_per_w,), jnp.int32),
            pltpu.VMEM((rows_per_w, d), jnp.float32),
            pltpu.SemaphoreType.DMA,
        ],
    )
    def combine(table_hbm, g_hbm, out_hbm, idx_v, rows_v, sem):
        wid = lax.axis_index("s") * 2 + lax.axis_index("c")
        base = wid * rows_per_w
        pltpu.sync_copy(g_hbm.at[pl.ds(base, rows_per_w)], idx_v)
        pltpu.async_copy(table_hbm.at[idx_v], rows_v, sem).wait()
        pltpu.sync_copy(rows_v, out_hbm.at[pl.ds(base, rows_per_w)])

    return combine


def kernel(x, Wsw, bsw, W1, b1, W2, b2):
    b, s, d = x.shape
    T = b * s
    E = Wsw.shape[1]
    FF = W1.shape[2]
    C = int(T * 1.25 / E)
    x_chunk = T // E
    pitch = C + x_chunk

    xt = x.reshape(T, d)
    xtb = xt.astype(jnp.bfloat16)

    g_col, counts, psum, nd = pl.pallas_call(
        functools.partial(_router_kernel, n_experts=E, capacity=C,
                          row_pitch=pitch, x_chunk=x_chunk),
        out_shape=(
            jax.ShapeDtypeStruct((T, 1), jnp.int32),
            jax.ShapeDtypeStruct((1, E), jnp.float32),
            jax.ShapeDtypeStruct((1, E), jnp.float32),
            jax.ShapeDtypeStruct((1, 1), jnp.int32),
        ),
    )(xtb, Wsw, bsw.reshape(1, E))

    g_row = g_col.reshape(1, T)

    table = pl.pallas_call(
        functools.partial(_expert_kernel, capacity=C, row_pitch=pitch),
        grid=(E,),
        in_specs=[
            pl.BlockSpec((1, T), lambda e: (0, 0)),
            pl.BlockSpec((T, d), lambda e: (0, 0)),
            pl.BlockSpec((x_chunk, d), lambda e: (e, 0)),
            pl.BlockSpec((None, d, FF), lambda e: (e, 0, 0)),
            pl.BlockSpec((None, 1, FF), lambda e: (e, 0, 0)),
            pl.BlockSpec((None, FF, d), lambda e: (e, 0, 0)),
            pl.BlockSpec((None, 1, d), lambda e: (e, 0, 0)),
        ],
        out_specs=pl.BlockSpec((None, pitch, d), lambda e: (e, 0, 0)),
        out_shape=jax.ShapeDtypeStruct((E, pitch, d), jnp.float32),
    )(g_row, xtb, xt, W1, b1.reshape(E, 1, FF), W2, b2.reshape(E, 1, d))

    table2 = table.reshape(E * pitch, d)

    out_t = _make_sc_combine(T, d, 32)(table2, g_col.reshape(T))

    out = out_t.reshape(b, s, d)
    return (out, counts.reshape(E), psum.reshape(E), nd.reshape(()))


# speedup vs baseline: 1.1953x; 1.0021x over previous
import functools

import jax
import jax.numpy as jnp
from jax import lax
from jax.experimental import pallas as pl
from jax.experimental.pallas import tpu as pltpu
from jax.experimental.pallas import tpu_sc as plsc


def _router_kernel(x_ref, w_ref, b_ref, g_ref, counts_ref, psum_ref, nd_ref,
                   *, n_experts, capacity, row_pitch, x_chunk):
    xv = x_ref[...]
    T = xv.shape[0]
    logits = jnp.dot(xv, w_ref[...].astype(jnp.bfloat16),
                     preferred_element_type=jnp.float32) + b_ref[...]
    mx = jnp.max(logits, axis=1, keepdims=True)
    el = jnp.exp(logits - mx)
    probs = el / jnp.sum(el, axis=1, keepdims=True)
    psum_ref[...] = jnp.sum(probs, axis=0, keepdims=True)

    eidx = lax.broadcasted_iota(jnp.int32, (T, n_experts), 1)
    is_max = logits == mx
    route = jnp.min(jnp.where(is_max, eidx, n_experts), axis=1, keepdims=True)

    onehot = (eidx == route).astype(jnp.float32)
    counts_ref[...] = jnp.sum(onehot, axis=0, keepdims=True)

    cb = 256
    r_io = lax.broadcasted_iota(jnp.int32, (cb, cb), 0)
    c_io = lax.broadcasted_iota(jnp.int32, (cb, cb), 1)
    tril = (r_io >= c_io).astype(jnp.bfloat16)
    onehot_b = onehot.astype(jnp.bfloat16)
    carry = jnp.zeros((1, n_experts), jnp.float32)
    pos_chunks = []
    for i in range(T // cb):
        oc = onehot_b[i * cb:(i + 1) * cb, :]
        csum = jnp.dot(tril, oc, preferred_element_type=jnp.float32) + carry
        carry = csum[cb - 1:cb, :]
        pos_chunks.append(jnp.sum((csum - 1.0)
                                  * onehot[i * cb:(i + 1) * cb, :],
                                  axis=1, keepdims=True))
    pos = jnp.concatenate(pos_chunks, axis=0)
    pos_i = jnp.round(pos).astype(jnp.int32)
    keep = pos_i < capacity
    tk = lax.broadcasted_iota(jnp.int32, (T, 1), 0)
    g_ref[...] = jnp.where(
        keep,
        route * row_pitch + pos_i,
        (tk // x_chunk) * row_pitch + capacity + tk % x_chunk)
    nd_ref[...] = T - jnp.sum(keep.astype(jnp.int32), keepdims=True)


def _expert_kernel(g_ref, x_ref, xf_ref, w1_ref, b1_ref, w2_ref, b2_ref,
                   y_ref, *, capacity, row_pitch):
    e = pl.program_id(0)
    gv = g_ref[...]
    cidx = lax.broadcasted_iota(jnp.int32, (capacity, gv.shape[1]), 0)
    m = (gv == e * row_pitch + cidx).astype(jnp.bfloat16)
    buf = jnp.dot(m, x_ref[...], preferred_element_type=jnp.float32)

    h = jnp.dot(buf, w1_ref[...],
                preferred_element_type=jnp.float32) + b1_ref[...]
    h = 0.5 * h * (1.0 + lax.erf(h * 0.7071067811865476))
    y_ref[0:capacity, :] = (jnp.dot(h, w2_ref[...],
                                    preferred_element_type=jnp.float32)
                            + b2_ref[...])
    y_ref[capacity:row_pitch, :] = xf_ref[...]


def _make_sc_combine(T, d, n_workers):
    rows_per_w = T // n_workers
    mesh = plsc.VectorSubcoreMesh(core_axis_name="c", subcore_axis_name="s")

    @functools.partial(
        pl.kernel, mesh=mesh,
        out_type=jax.ShapeDtypeStruct((T, d), jnp.float32),
        scratch_types=[
            pltpu.VMEM((rows_per_w,), jnp.int32),
            pltpu.VMEM((rows_per_w, d), jnp.float32),
            pltpu.SemaphoreType.DMA,
        ],
    )
    def combine(table_hbm, g_hbm, out_hbm, idx_v, rows_v, sem):
        wid = lax.axis_index("s") * 2 + lax.axis_index("c")
        base = wid * rows_per_w
        pltpu.sync_copy(g_hbm.at[pl.ds(base, rows_per_w)], idx_v)
        pltpu.async_copy(table_hbm.at[idx_v], rows_v, sem).wait()
        pltpu.sync_copy(rows_v, out_hbm.at[pl.ds(base, rows_per_w)])

    return combine


def kernel(x, Wsw, bsw, W1, b1, W2, b2):
    b, s, d = x.shape
    T = b * s
    E = Wsw.shape[1]
    FF = W1.shape[2]
    C = int(T * 1.25 / E)
    x_chunk = T // E
    pitch = C + x_chunk

    xt = x.reshape(T, d)
    xtb = xt.astype(jnp.bfloat16)

    g_col, counts, psum, nd = pl.pallas_call(
        functools.partial(_router_kernel, n_experts=E, capacity=C,
                          row_pitch=pitch, x_chunk=x_chunk),
        out_shape=(
            jax.ShapeDtypeStruct((T, 1), jnp.int32),
            jax.ShapeDtypeStruct((1, E), jnp.float32),
            jax.ShapeDtypeStruct((1, E), jnp.float32),
            jax.ShapeDtypeStruct((1, 1), jnp.int32),
        ),
    )(xtb, Wsw, bsw.reshape(1, E))

    g_row = g_col.reshape(1, T)

    table = pl.pallas_call(
        functools.partial(_expert_kernel, capacity=C, row_pitch=pitch),
        grid=(E,),
        in_specs=[
            pl.BlockSpec((1, T), lambda e: (0, 0)),
            pl.BlockSpec((T, d), lambda e: (0, 0)),
            pl.BlockSpec((x_chunk, d), lambda e: (e, 0)),
            pl.BlockSpec((None, d, FF), lambda e: (e, 0, 0)),
            pl.BlockSpec((None, 1, FF), lambda e: (e, 0, 0)),
            pl.BlockSpec((None, FF, d), lambda e: (e, 0, 0)),
            pl.BlockSpec((None, 1, d), lambda e: (e, 0, 0)),
        ],
        out_specs=pl.BlockSpec((None, pitch, d), lambda e: (e, 0, 0)),
        out_shape=jax.ShapeDtypeStruct((E, pitch, d), jnp.float32),
    )(g_row, xtb, xt, W1, b1.reshape(E, 1, FF), W2, b2.reshape(E, 1, d))

    table2 = table.reshape(E * pitch, d)

    out_t = _make_sc_combine(T, d, 32)(table2, g_col.reshape(T))

    out = out_t.reshape(b, s, d)
    return (out, counts.reshape(E), psum.reshape(E), nd.reshape(()))
